# baseline (speedup 1.0000x reference)
import jax
import jax.numpy as jnp
from jax import lax
from jax.experimental import pallas as pl
from jax.experimental.pallas import tpu as pltpu
from jax.experimental.pallas import tpu_sc as plsc

NUM_FIELDS = 26
FIELD_DIM = 38461
TOTAL_ROWS = NUM_FIELDS * FIELD_DIM
BATCH = 16384
LANES = 16
NUM_CORES = 2
NUM_SUBCORES = 16
SC_BATCH = BATCH // NUM_CORES
TILE_BATCH = SC_BATCH // NUM_SUBCORES
VECS_PER_TILE = TILE_BATCH // LANES
UNROLL = 4
GATHER_ITERS = SC_BATCH // (LANES * UNROLL)
FIELD_PAD = 38464


def _body(xt_hbm, w_hbm, b_hbm, out_hbm,
          tbl_a, tbl_b, idx_a, idx_b, part_a,
          red_v, out_v, bias_v, sem_ta, sem_tb, sem_ia, sem_ib,
          sem_pa, sem_r, part_sh):
    c = lax.axis_index("c")
    s = lax.axis_index("s")
    sc_base = c * SC_BATCH
    gbase = sc_base + s * TILE_BATCH

    f1 = s
    f2 = s + NUM_SUBCORES
    has_f2 = f2 < NUM_FIELDS

    def issue_field(f, tbl_v, idx_v, sem_t, sem_i):
        cp_t = pltpu.make_async_copy(w_hbm.at[f, :], tbl_v, sem_t)
        cp_t.start()
        cp_i = pltpu.make_async_copy(
            xt_hbm.at[f, pl.ds(sc_base, SC_BATCH)], idx_v, sem_i)
        cp_i.start()
        return cp_t, cp_i

    def gather_field(f, tbl_v, idx_v, part_v):
        def g_body(k, carry):
            base = k * (LANES * UNROLL)
            for u in range(UNROLL):
                iv = idx_v[pl.ds(base + u * LANES, LANES)]
                part_v[pl.ds(base + u * LANES, LANES)] = (
                    plsc.load_gather(tbl_v, [iv]))
            return carry

        lax.fori_loop(0, GATHER_ITERS, g_body, 0)

    cp_t1, cp_i1 = issue_field(f1, tbl_a, idx_a, sem_ta, sem_ia)

    @pl.when(has_f2)
    def _():
        issue_field(f2, tbl_b, idx_b, sem_tb, sem_ib)

    cp_t1.wait()
    cp_i1.wait()
    gather_field(f1, tbl_a, idx_a, part_a)
    cp_p1 = pltpu.make_async_copy(part_a, part_sh.at[f1, :], sem_pa)
    cp_p1.start()

    @pl.when(has_f2)
    def _():
        pltpu.make_async_copy(w_hbm.at[f2, :], tbl_b, sem_tb).wait()
        pltpu.make_async_copy(
            xt_hbm.at[f2, pl.ds(sc_base, SC_BATCH)], idx_b, sem_ib).wait()

    cp_p1.wait()

    @pl.when(has_f2)
    def _():
        gather_field(f2, tbl_b, idx_b, part_a)
        pltpu.sync_copy(part_a, part_sh.at[f2, :])

    plsc.subcore_barrier()

    pltpu.sync_copy(b_hbm, bias_v)
    cps = []
    for f in range(NUM_FIELDS):
        cp = pltpu.make_async_copy(
            part_sh.at[f, pl.ds(s * TILE_BATCH, TILE_BATCH)],
            red_v.at[f, :], sem_r)
        cp.start()
        cps.append(cp)
    for cp in cps:
        cp.wait()
    bias_vec = bias_v[...]

    def r_body(k, carry):
        acc = red_v[0, pl.ds(k * LANES, LANES)]
        for f in range(1, NUM_FIELDS):
            acc = acc + red_v[f, pl.ds(k * LANES, LANES)]
        out_v[pl.ds(k * LANES, LANES)] = acc + bias_vec
        return carry

    lax.fori_loop(0, VECS_PER_TILE, r_body, 0)
    pltpu.sync_copy(out_v, out_hbm.at[pl.ds(gbase, TILE_BATCH)])


@jax.jit
def _features_linear(xt, w, b):
    mesh = plsc.VectorSubcoreMesh(core_axis_name="c", subcore_axis_name="s")
    return pl.kernel(
        _body,
        out_type=jax.ShapeDtypeStruct((BATCH,), jnp.float32),
        mesh=mesh,
        compiler_params=pltpu.CompilerParams(
            needs_layout_passes=False, use_tc_tiling_on_sc=False),
        scratch_types=[
            pltpu.VMEM((FIELD_PAD,), jnp.float32),
            pltpu.VMEM((FIELD_PAD,), jnp.float32),
            pltpu.VMEM((SC_BATCH,), jnp.int32),
            pltpu.VMEM((SC_BATCH,), jnp.int32),
            pltpu.VMEM((SC_BATCH,), jnp.float32),
            pltpu.VMEM((NUM_FIELDS, TILE_BATCH), jnp.float32),
            pltpu.VMEM((TILE_BATCH,), jnp.float32),
            pltpu.VMEM((LANES,), jnp.float32),
            pltpu.SemaphoreType.DMA,
            pltpu.SemaphoreType.DMA,
            pltpu.SemaphoreType.DMA,
            pltpu.SemaphoreType.DMA,
            pltpu.SemaphoreType.DMA,
            pltpu.SemaphoreType.DMA,
            pltpu.VMEM_SHARED((NUM_FIELDS, SC_BATCH), jnp.float32),
        ],
    )(xt, w, b)


def kernel(x, fc_weight, bias):
    w2 = jnp.zeros((NUM_FIELDS, FIELD_PAD), jnp.float32)
    for f in range(NUM_FIELDS):
        w2 = lax.dynamic_update_slice(
            w2, fc_weight[f * FIELD_DIM:(f + 1) * FIELD_DIM].T, (f, 0))
    b16 = jnp.broadcast_to(bias, (LANES,))
    return _features_linear(x.T, w2, b16).reshape(BATCH, 1)


# speedup vs baseline: 1.9050x; 1.9050x over previous
import jax
import jax.numpy as jnp
from jax import lax
from jax.experimental import pallas as pl
from jax.experimental.pallas import tpu as pltpu
from jax.experimental.pallas import tpu_sc as plsc

NUM_FIELDS = 26
FIELD_DIM = 38461
TOTAL_ROWS = NUM_FIELDS * FIELD_DIM
BATCH = 16384
LANES = 16
NUM_CORES = 2
NUM_SUBCORES = 16
SC_BATCH = BATCH // NUM_CORES
TILE_BATCH = SC_BATCH // NUM_SUBCORES
VECS_PER_TILE = TILE_BATCH // LANES
UNROLL = 4
GATHER_ITERS = SC_BATCH // (LANES * UNROLL)
FIELD_PAD = 38464


def _body(xt_hbm, w_hbm, b_hbm, out_hbm,
          tbl_a, tbl_b, idx_a, idx_b, part_a,
          red_v, out_v, bias_v, sem_ta, sem_tb, sem_ia, sem_ib,
          sem_pa, sem_r, part_sh):
    c = lax.axis_index("c")
    s = lax.axis_index("s")
    sc_base = c * SC_BATCH
    gbase = sc_base + s * TILE_BATCH

    f1 = s
    f2 = s + NUM_SUBCORES
    has_f2 = f2 < NUM_FIELDS

    def issue_field(f, tbl_v, idx_v, sem_t, sem_i):
        cp_t = pltpu.make_async_copy(w_hbm.at[f, :], tbl_v, sem_t)
        cp_t.start()
        cp_i = pltpu.make_async_copy(
            xt_hbm.at[f, pl.ds(sc_base, SC_BATCH)], idx_v, sem_i)
        cp_i.start()
        return cp_t, cp_i

    def gather_field(f, tbl_v, idx_v, part_v):
        def g_body(k, carry):
            base = k * (LANES * UNROLL)
            for u in range(UNROLL):
                iv = idx_v[pl.ds(base + u * LANES, LANES)]
                part_v[pl.ds(base + u * LANES, LANES)] = (
                    plsc.load_gather(tbl_v, [iv]))
            return carry

        lax.fori_loop(0, GATHER_ITERS, g_body, 0)

    cp_t1, cp_i1 = issue_field(f1, tbl_a, idx_a, sem_ta, sem_ia)

    @pl.when(has_f2)
    def _():
        issue_field(f2, tbl_b, idx_b, sem_tb, sem_ib)

    cp_t1.wait()
    cp_i1.wait()
    gather_field(f1, tbl_a, idx_a, part_a)
    cp_p1 = pltpu.make_async_copy(part_a, part_sh.at[f1, :], sem_pa)
    cp_p1.start()

    @pl.when(has_f2)
    def _():
        pltpu.make_async_copy(w_hbm.at[f2, :], tbl_b, sem_tb).wait()
        pltpu.make_async_copy(
            xt_hbm.at[f2, pl.ds(sc_base, SC_BATCH)], idx_b, sem_ib).wait()

    cp_p1.wait()

    @pl.when(has_f2)
    def _():
        gather_field(f2, tbl_b, idx_b, part_a)
        pltpu.sync_copy(part_a, part_sh.at[f2, :])

    plsc.subcore_barrier()

    pltpu.sync_copy(b_hbm, bias_v)
    cps = []
    for f in range(NUM_FIELDS):
        cp = pltpu.make_async_copy(
            part_sh.at[f, pl.ds(s * TILE_BATCH, TILE_BATCH)],
            red_v.at[f, :], sem_r)
        cp.start()
        cps.append(cp)
    for cp in cps:
        cp.wait()
    bias_vec = bias_v[...]

    def r_body(k, carry):
        acc = red_v[0, pl.ds(k * LANES, LANES)]
        for f in range(1, NUM_FIELDS):
            acc = acc + red_v[f, pl.ds(k * LANES, LANES)]
        out_v[pl.ds(k * LANES, LANES)] = acc + bias_vec
        return carry

    lax.fori_loop(0, VECS_PER_TILE, r_body, 0)
    pltpu.sync_copy(out_v, out_hbm.at[pl.ds(gbase, TILE_BATCH)])


@jax.jit
def _features_linear(xt, w, b):
    mesh = plsc.VectorSubcoreMesh(core_axis_name="c", subcore_axis_name="s")
    return pl.kernel(
        _body,
        out_type=jax.ShapeDtypeStruct((BATCH,), jnp.float32),
        mesh=mesh,
        compiler_params=pltpu.CompilerParams(
            needs_layout_passes=False, use_tc_tiling_on_sc=False),
        scratch_types=[
            pltpu.VMEM((FIELD_PAD,), jnp.float32),
            pltpu.VMEM((FIELD_PAD,), jnp.float32),
            pltpu.VMEM((SC_BATCH,), jnp.int32),
            pltpu.VMEM((SC_BATCH,), jnp.int32),
            pltpu.VMEM((SC_BATCH,), jnp.float32),
            pltpu.VMEM((NUM_FIELDS, TILE_BATCH), jnp.float32),
            pltpu.VMEM((TILE_BATCH,), jnp.float32),
            pltpu.VMEM((LANES,), jnp.float32),
            pltpu.SemaphoreType.DMA,
            pltpu.SemaphoreType.DMA,
            pltpu.SemaphoreType.DMA,
            pltpu.SemaphoreType.DMA,
            pltpu.SemaphoreType.DMA,
            pltpu.SemaphoreType.DMA,
            pltpu.VMEM_SHARED((NUM_FIELDS, SC_BATCH), jnp.float32),
        ],
    )(xt, w, b)


def kernel(x, fc_weight, bias):
    wt = fc_weight.T
    rows = [lax.slice(wt, (0, f * FIELD_DIM), (1, (f + 1) * FIELD_DIM))
            for f in range(NUM_FIELDS)]
    w2 = jnp.pad(jnp.concatenate(rows, axis=0),
                 ((0, 0), (0, FIELD_PAD - FIELD_DIM)))
    b16 = jnp.broadcast_to(bias, (LANES,))
    return _features_linear(x.T, w2, b16).reshape(BATCH, 1)


# speedup vs baseline: 1.9170x; 1.0063x over previous
import jax
import jax.numpy as jnp
from jax import lax
from jax.experimental import pallas as pl
from jax.experimental.pallas import tpu as pltpu
from jax.experimental.pallas import tpu_sc as plsc

NUM_FIELDS = 26
FIELD_DIM = 38461
TOTAL_ROWS = NUM_FIELDS * FIELD_DIM
BATCH = 16384
LANES = 16
NUM_CORES = 2
NUM_SUBCORES = 16
SC_BATCH = BATCH // NUM_CORES
TILE_BATCH = SC_BATCH // NUM_SUBCORES
VECS_PER_TILE = TILE_BATCH // LANES
UNROLL = 8
GATHER_ITERS = SC_BATCH // (LANES * UNROLL)
FIELD_PAD = 38464


def _body(xt_hbm, w_hbm, b_hbm, out_hbm,
          tbl_a, tbl_b, idx_a, idx_b, part_a,
          red_v, out_v, bias_v, sem_ta, sem_tb, sem_ia, sem_ib,
          sem_pa, sem_r, part_sh):
    c = lax.axis_index("c")
    s = lax.axis_index("s")
    sc_base = c * SC_BATCH
    gbase = sc_base + s * TILE_BATCH

    f1 = s
    f2 = s + NUM_SUBCORES
    has_f2 = f2 < NUM_FIELDS

    def issue_field(f, tbl_v, idx_v, sem_t, sem_i):
        cp_t = pltpu.make_async_copy(w_hbm.at[f, :], tbl_v, sem_t)
        cp_t.start()
        cp_i = pltpu.make_async_copy(
            xt_hbm.at[f, pl.ds(sc_base, SC_BATCH)], idx_v, sem_i)
        cp_i.start()
        return cp_t, cp_i

    def gather_field(f, tbl_v, idx_v, part_v):
        def g_body(k, carry):
            base = k * (LANES * UNROLL)
            for u in range(UNROLL):
                iv = idx_v[pl.ds(base + u * LANES, LANES)]
                part_v[pl.ds(base + u * LANES, LANES)] = (
                    plsc.load_gather(tbl_v, [iv]))
            return carry

        lax.fori_loop(0, GATHER_ITERS, g_body, 0)

    cp_t1, cp_i1 = issue_field(f1, tbl_a, idx_a, sem_ta, sem_ia)

    @pl.when(has_f2)
    def _():
        issue_field(f2, tbl_b, idx_b, sem_tb, sem_ib)

    cp_t1.wait()
    cp_i1.wait()
    gather_field(f1, tbl_a, idx_a, part_a)
    cp_p1 = pltpu.make_async_copy(part_a, part_sh.at[f1, :], sem_pa)
    cp_p1.start()

    @pl.when(has_f2)
    def _():
        pltpu.make_async_copy(w_hbm.at[f2, :], tbl_b, sem_tb).wait()
        pltpu.make_async_copy(
            xt_hbm.at[f2, pl.ds(sc_base, SC_BATCH)], idx_b, sem_ib).wait()

    @pl.when(has_f2)
    def _():
        gather_field(f2, tbl_b, idx_b, red_v.at[pl.ds(0, SC_BATCH)])
        pltpu.sync_copy(red_v.at[pl.ds(0, SC_BATCH)], part_sh.at[f2, :])

    cp_p1.wait()

    plsc.subcore_barrier()

    pltpu.sync_copy(b_hbm, bias_v)
    cps = []
    for f in range(NUM_FIELDS):
        cp = pltpu.make_async_copy(
            part_sh.at[f, pl.ds(s * TILE_BATCH, TILE_BATCH)],
            red_v.at[pl.ds(f * TILE_BATCH, TILE_BATCH)], sem_r)
        cp.start()
        cps.append(cp)
    for cp in cps:
        cp.wait()
    bias_vec = bias_v[...]

    def r_body(k, carry):
        acc = red_v[pl.ds(k * LANES, LANES)]
        for f in range(1, NUM_FIELDS):
            acc = acc + red_v[pl.ds(f * TILE_BATCH + k * LANES, LANES)]
        out_v[pl.ds(k * LANES, LANES)] = acc + bias_vec
        return carry

    lax.fori_loop(0, VECS_PER_TILE, r_body, 0)
    pltpu.sync_copy(out_v, out_hbm.at[pl.ds(gbase, TILE_BATCH)])


@jax.jit
def _features_linear(xt, w, b):
    mesh = plsc.VectorSubcoreMesh(core_axis_name="c", subcore_axis_name="s")
    return pl.kernel(
        _body,
        out_type=jax.ShapeDtypeStruct((BATCH,), jnp.float32),
        mesh=mesh,
        compiler_params=pltpu.CompilerParams(
            needs_layout_passes=False, use_tc_tiling_on_sc=False),
        scratch_types=[
            pltpu.VMEM((FIELD_PAD,), jnp.float32),
            pltpu.VMEM((FIELD_PAD,), jnp.float32),
            pltpu.VMEM((SC_BATCH,), jnp.int32),
            pltpu.VMEM((SC_BATCH,), jnp.int32),
            pltpu.VMEM((SC_BATCH,), jnp.float32),
            pltpu.VMEM((NUM_FIELDS * TILE_BATCH,), jnp.float32),
            pltpu.VMEM((TILE_BATCH,), jnp.float32),
            pltpu.VMEM((LANES,), jnp.float32),
            pltpu.SemaphoreType.DMA,
            pltpu.SemaphoreType.DMA,
            pltpu.SemaphoreType.DMA,
            pltpu.SemaphoreType.DMA,
            pltpu.SemaphoreType.DMA,
            pltpu.SemaphoreType.DMA,
            pltpu.VMEM_SHARED((NUM_FIELDS, SC_BATCH), jnp.float32),
        ],
    )(xt, w, b)


def kernel(x, fc_weight, bias):
    wt = fc_weight.T
    rows = [lax.slice(wt, (0, f * FIELD_DIM), (1, (f + 1) * FIELD_DIM))
            for f in range(NUM_FIELDS)]
    w2 = jnp.pad(jnp.concatenate(rows, axis=0),
                 ((0, 0), (0, FIELD_PAD - FIELD_DIM)))
    b16 = jnp.broadcast_to(bias, (LANES,))
    return _features_linear(x.T, w2, b16).reshape(BATCH, 1)
